# baseline (speedup 1.0000x reference)
import functools

import jax
import jax.numpy as jnp
from jax import lax
from jax.experimental import pallas as pl
from jax.experimental.pallas import tpu as pltpu
from jax.experimental.pallas import tpu_sc as plsc

_N = 10000
_E = 160000
_DIN = 256
_DOUT = 50
_DP = 64
_NC = 2
_NS = 16
_NW = _NC * _NS
_EPW = _E // _NW
_C = 125
_NCH = _EPW // _C
_NP = 10240
_RPT = _NP // _NS


def _ymat_body(x_ref, wl_ref, y_ref):
    xb = x_ref[...].astype(jnp.bfloat16)
    yv = lax.dot_general(xb, wl_ref[...].astype(jnp.bfloat16),
                         (((1,), (1,)), ((), ())),
                         preferred_element_type=jnp.float32)
    y_ref[...] = jnp.zeros_like(y_ref)
    y_ref[:, :_DOUT] = yv.astype(jnp.bfloat16)
    y_ref[:, _DOUT:_DOUT + 1] = jnp.ones((y_ref.shape[0], 1), jnp.bfloat16)


def _ymat(x, wl):
    blk = 5000
    grid = _N // blk
    return pl.pallas_call(
        _ymat_body,
        grid=(grid,),
        in_specs=[
            pl.BlockSpec((blk, _DIN), lambda i: (i, 0)),
            pl.BlockSpec((_DOUT, _DIN), lambda i: (0, 0)),
        ],
        out_specs=pl.BlockSpec((blk, _DP), lambda i: (i, 0)),
        out_shape=jax.ShapeDtypeStruct((_N, _DP), jnp.bfloat16),
    )(x, wl)


def _zmat_body(x_ref, wr_ref, b_ref, z_ref):
    xb = x_ref[...].astype(jnp.bfloat16)
    zv = lax.dot_general(xb, wr_ref[...].astype(jnp.bfloat16),
                         (((1,), (1,)), ((), ())),
                         preferred_element_type=jnp.float32)
    z_ref[...] = jnp.zeros_like(z_ref)
    z_ref[:, :_DOUT] = zv + b_ref[...][None, :]


def _zmat(x, wr, b):
    blk = 2000
    grid = _N // blk
    return pl.pallas_call(
        _zmat_body,
        grid=(grid,),
        in_specs=[
            pl.BlockSpec((blk, _DIN), lambda i: (i, 0)),
            pl.BlockSpec((_DOUT, _DIN), lambda i: (0, 0)),
            pl.BlockSpec((_DOUT,), lambda i: (0,)),
        ],
        out_specs=pl.BlockSpec((blk, _DP), lambda i: (i, 0)),
        out_shape=jax.ShapeDtypeStruct((_N, _DP), jnp.float32),
    )(x, wr, b)


def _sc_agg_body(y_hbm, ei_hbm, zero_hbm, out_hbm,
                 src_v, dst_v, rows0, rows1, y_sh, acc_sh, gsem0, gsem1):
    cid = lax.axis_index("c")
    sid = lax.axis_index("s")
    wid = cid * _NS + sid
    r0 = pl.multiple_of(sid * _RPT, 8)
    ys = _N // _NS
    pltpu.sync_copy(y_hbm.at[pl.ds(sid * ys, ys)], y_sh.at[pl.ds(sid * ys, ys)])
    pltpu.sync_copy(zero_hbm, acc_sh.at[pl.ds(r0, _RPT)])
    pltpu.sync_copy(ei_hbm.at[0, wid], src_v)
    pltpu.sync_copy(ei_hbm.at[1, wid], dst_v)
    plsc.subcore_barrier()

    def _gather(j, buf, sem):
        pltpu.async_copy(y_sh.at[src_v.at[j]], buf, sem)

    def _gwait(buf, sem):
        pltpu.make_async_copy(y_sh.at[src_v.at[0]], buf, sem).wait()

    def _scatter(j, buf):
        pltpu.sync_copy(buf, acc_sh.at[dst_v.at[j]], add=True)

    _gather(0, rows0, gsem0)

    @pl.loop(0, _NCH - 2, step=2)
    def _(j):
        _gather(j + 1, rows1, gsem1)
        _gwait(rows0, gsem0)
        _scatter(j, rows0)
        _gather(j + 2, rows0, gsem0)
        _gwait(rows1, gsem1)
        _scatter(j + 1, rows1)

    _gather(_NCH - 1, rows1, gsem1)
    _gwait(rows0, gsem0)
    _scatter(_NCH - 2, rows0)
    _gwait(rows1, gsem1)
    _scatter(_NCH - 1, rows1)

    plsc.subcore_barrier()
    pltpu.sync_copy(acc_sh.at[pl.ds(r0, _RPT)],
                    out_hbm.at[cid, pl.ds(r0, _RPT)])


@functools.lru_cache(maxsize=1)
def _sc_agg_fn():
    return pl.kernel(
        _sc_agg_body,
        out_type=jax.ShapeDtypeStruct((_NC, _NP, _DP), jnp.bfloat16),
        mesh=plsc.VectorSubcoreMesh(core_axis_name="c", subcore_axis_name="s",
                                    num_cores=_NC, num_subcores=_NS),
        compiler_params=pltpu.CompilerParams(use_tc_tiling_on_sc=False,
                                             disable_bounds_checks=True),
        scratch_types=(
            [pltpu.VMEM((_NCH, _C), jnp.int32)] * 2
            + [pltpu.VMEM((_C, _DP), jnp.bfloat16)] * 2
            + [pltpu.VMEM_SHARED((_N, _DP), jnp.bfloat16),
               pltpu.VMEM_SHARED((_NP, _DP), jnp.bfloat16)]
            + [pltpu.SemaphoreType.DMA] * 2
        ),
    )


_FBLK = 5000


def _finish_body(p_hbm, z_ref, o_ref, p0_v, p1_v, psem0, psem1):
    i = pl.program_id(0)
    cp0 = pltpu.make_async_copy(p_hbm.at[0, pl.ds(i * _FBLK, _FBLK)],
                                p0_v, psem0)
    cp1 = pltpu.make_async_copy(p_hbm.at[1, pl.ds(i * _FBLK, _FBLK)],
                                p1_v, psem1)
    cp0.start()
    cp1.start()
    cp0.wait()
    cp1.wait()
    agg = p0_v[...].astype(jnp.float32) + p1_v[...].astype(jnp.float32)
    cnt = agg[:, _DOUT:_DOUT + 1]
    mean = agg / jnp.maximum(cnt, 1.0)
    t = mean + z_ref[...]
    col = lax.broadcasted_iota(jnp.int32, t.shape, 1)
    t = jnp.where(col < _DOUT, t, 0.0)
    ss = jnp.sum(t * t, axis=1, keepdims=True)
    nrm = jnp.maximum(jnp.sqrt(ss), 1e-12)
    o_ref[...] = (t / nrm)[:, :_DOUT]


def _finish(partial, z):
    blk = _FBLK
    grid = _N // blk
    return pl.pallas_call(
        _finish_body,
        grid=(grid,),
        in_specs=[
            pl.BlockSpec(memory_space=pl.ANY),
            pl.BlockSpec((blk, _DP), lambda i: (i, 0)),
        ],
        out_specs=pl.BlockSpec((blk, _DOUT), lambda i: (i, 0)),
        out_shape=jax.ShapeDtypeStruct((_N, _DOUT), jnp.float32),
        scratch_shapes=[
            pltpu.VMEM((blk, _DP), jnp.bfloat16),
            pltpu.VMEM((blk, _DP), jnp.bfloat16),
            pltpu.SemaphoreType.DMA,
            pltpu.SemaphoreType.DMA,
        ],
    )(partial, z)


def kernel(x, edge_index, W_l, b_l, W_r):
    ei = edge_index.reshape(2, _NW, _NCH, _C)
    zeros = jnp.zeros((_RPT, _DP), jnp.bfloat16)

    y = _ymat(x, W_l)
    partial = _sc_agg_fn()(y, ei, zeros)
    z = _zmat(x, W_r, b_l)
    return _finish(partial, z)


# speedup vs baseline: 1.0426x; 1.0426x over previous
import functools

import jax
import jax.numpy as jnp
from jax import lax
from jax.experimental import pallas as pl
from jax.experimental.pallas import tpu as pltpu
from jax.experimental.pallas import tpu_sc as plsc

_N = 10000
_E = 160000
_DIN = 256
_DOUT = 50
_DP = 64
_NC = 2
_NS = 16
_NW = _NC * _NS
_EPW = _E // _NW
_C = 125
_NCH = _EPW // _C
_NP = 10240
_RPT = _NP // _NS


def _ymat_body(x_ref, wl_ref, y_ref):
    xb = x_ref[...].astype(jnp.bfloat16)
    yv = lax.dot_general(xb, wl_ref[...].astype(jnp.bfloat16),
                         (((1,), (1,)), ((), ())),
                         preferred_element_type=jnp.float32)
    y_ref[...] = jnp.zeros_like(y_ref)
    y_ref[:, :_DOUT] = yv.astype(jnp.bfloat16)
    y_ref[:, _DOUT:_DOUT + 1] = jnp.ones((y_ref.shape[0], 1), jnp.bfloat16)


def _ymat(x, wl):
    blk = 5000
    grid = _N // blk
    return pl.pallas_call(
        _ymat_body,
        grid=(grid,),
        in_specs=[
            pl.BlockSpec((blk, _DIN), lambda i: (i, 0)),
            pl.BlockSpec((_DOUT, _DIN), lambda i: (0, 0)),
        ],
        out_specs=pl.BlockSpec((blk, _DP), lambda i: (i, 0)),
        out_shape=jax.ShapeDtypeStruct((_N, _DP), jnp.bfloat16),
    )(x, wl)


def _zmat_body(x_ref, wr_ref, b_ref, z_ref):
    xb = x_ref[...].astype(jnp.bfloat16)
    zv = lax.dot_general(xb, wr_ref[...].astype(jnp.bfloat16),
                         (((1,), (1,)), ((), ())),
                         preferred_element_type=jnp.float32)
    z_ref[...] = jnp.zeros_like(z_ref)
    z_ref[:, :_DOUT] = zv + b_ref[...][None, :]


def _zmat(x, wr, b):
    blk = 2000
    grid = _N // blk
    return pl.pallas_call(
        _zmat_body,
        grid=(grid,),
        in_specs=[
            pl.BlockSpec((blk, _DIN), lambda i: (i, 0)),
            pl.BlockSpec((_DOUT, _DIN), lambda i: (0, 0)),
            pl.BlockSpec((_DOUT,), lambda i: (0,)),
        ],
        out_specs=pl.BlockSpec((blk, _DP), lambda i: (i, 0)),
        out_shape=jax.ShapeDtypeStruct((_N, _DP), jnp.float32),
    )(x, wr, b)


def _sc_agg_body(y_hbm, ei_hbm, zero_hbm, out_hbm,
                 src_v, dst_v, rows0, rows1, y_sh, acc_sh, gsem0, gsem1):
    cid = lax.axis_index("c")
    sid = lax.axis_index("s")
    wid = cid * _NS + sid
    r0 = pl.multiple_of(sid * _RPT, 8)
    ys = _N // _NS
    pltpu.sync_copy(y_hbm.at[pl.ds(sid * ys, ys)], y_sh.at[pl.ds(sid * ys, ys)])
    pltpu.sync_copy(zero_hbm, acc_sh.at[pl.ds(r0, _RPT)])
    pltpu.sync_copy(ei_hbm.at[0, wid], src_v)
    pltpu.sync_copy(ei_hbm.at[1, wid], dst_v)
    plsc.subcore_barrier()

    def _gather(j, buf, sem):
        pltpu.async_copy(y_sh.at[src_v.at[j]], buf, sem)

    def _gwait(buf, sem):
        pltpu.make_async_copy(y_sh.at[src_v.at[0]], buf, sem).wait()

    def _scatter(j, buf):
        pltpu.sync_copy(buf, acc_sh.at[dst_v.at[j]], add=True)

    _gather(0, rows0, gsem0)

    @pl.loop(0, _NCH - 2, step=2)
    def _(j):
        _gather(j + 1, rows1, gsem1)
        _gwait(rows0, gsem0)
        _scatter(j, rows0)
        _gather(j + 2, rows0, gsem0)
        _gwait(rows1, gsem1)
        _scatter(j + 1, rows1)

    _gather(_NCH - 1, rows1, gsem1)
    _gwait(rows0, gsem0)
    _scatter(_NCH - 2, rows0)
    _gwait(rows1, gsem1)
    _scatter(_NCH - 1, rows1)

    plsc.subcore_barrier()
    pltpu.sync_copy(acc_sh.at[pl.ds(r0, _RPT)],
                    out_hbm.at[cid, pl.ds(r0, _RPT)])


@functools.lru_cache(maxsize=1)
def _sc_agg_fn():
    return pl.kernel(
        _sc_agg_body,
        out_type=jax.ShapeDtypeStruct((_NC, _NP, _DP), jnp.bfloat16),
        mesh=plsc.VectorSubcoreMesh(core_axis_name="c", subcore_axis_name="s",
                                    num_cores=_NC, num_subcores=_NS),
        compiler_params=pltpu.CompilerParams(use_tc_tiling_on_sc=False,
                                             disable_bounds_checks=True),
        scratch_types=(
            [pltpu.VMEM((_NCH, _C), jnp.int32)] * 2
            + [pltpu.VMEM((_C, _DP), jnp.bfloat16)] * 2
            + [pltpu.VMEM_SHARED((_N, _DP), jnp.bfloat16),
               pltpu.VMEM_SHARED((_NP, _DP), jnp.bfloat16)]
            + [pltpu.SemaphoreType.DMA] * 2
        ),
    )


def _finish_body(p_ref, z_ref, o_ref):
    agg = p_ref[0].astype(jnp.float32) + p_ref[1].astype(jnp.float32)
    cnt = agg[:, _DOUT:_DOUT + 1]
    mean = agg / jnp.maximum(cnt, 1.0)
    t = mean + z_ref[...]
    col = lax.broadcasted_iota(jnp.int32, t.shape, 1)
    t = jnp.where(col < _DOUT, t, 0.0)
    ss = jnp.sum(t * t, axis=1, keepdims=True)
    nrm = jnp.maximum(jnp.sqrt(ss), 1e-12)
    o_ref[...] = (t / nrm)[:, :_DOUT]


def _finish(partial, z):
    blk = 5000
    grid = _N // blk
    return pl.pallas_call(
        _finish_body,
        grid=(grid,),
        in_specs=[
            pl.BlockSpec((_NC, blk, _DP), lambda i: (0, i, 0)),
            pl.BlockSpec((blk, _DP), lambda i: (i, 0)),
        ],
        out_specs=pl.BlockSpec((blk, _DOUT), lambda i: (i, 0)),
        out_shape=jax.ShapeDtypeStruct((_N, _DOUT), jnp.float32),
    )(partial, z)


def kernel(x, edge_index, W_l, b_l, W_r):
    ei = edge_index.reshape(2, _NW, _NCH, _C)
    zeros = jnp.zeros((_RPT, _DP), jnp.bfloat16)

    y = _ymat(x, W_l)
    partial = _sc_agg_fn()(y, ei, zeros)
    z = _zmat(x, W_r, b_l)
    return _finish(partial, z)
